# baseline (speedup 1.0000x reference)
import functools

import numpy as np
import jax
import jax.numpy as jnp
from jax import lax
from jax.experimental import pallas as pl
from jax.experimental.pallas import tpu as pltpu
from jax.experimental.pallas import tpu_sc as plsc

_NUM_FIELDS = 26
_EMBED_DIM = 16
_BATCH = 16384
_FIELD_DIM = 100000
_NROWS = _FIELD_DIM * _NUM_FIELDS
_OFFSETS = np.arange(_NUM_FIELDS, dtype=np.int32) * _FIELD_DIM

_NC = 2
_NS = 16
_NW = _NC * _NS
_L = 16

_TC_FULL = _NROWS // 128
_TAILROWS = _NROWS - _TC_FULL * 128
_TCQ, _TCR = divmod(_TC_FULL, _NW)
_CRING = 8

_BPW = _BATCH // _NW
_CB = 16
_ROWS = _CB * _NUM_FIELDS
_NCHUNK = _BPW // _CB
_GSUB = 104
_NG = _ROWS // _GSUB
_XLEN = _BPW * _NUM_FIELDS
_RING = 2

_MESH = dict(core_axis_name="c", subcore_axis_name="s")


def _to_row_major(table):
    tT = table.T
    tail = table[_TC_FULL * 128:, :].reshape(-1)

    @functools.partial(
        pl.kernel,
        mesh=plsc.VectorSubcoreMesh(**_MESH),
        out_type=jax.ShapeDtypeStruct((_NROWS * _EMBED_DIM,), jnp.float32),
        compiler_params=pltpu.CompilerParams(
            needs_layout_passes=False, use_tc_tiling_on_sc=True
        ),
        scratch_types=(
            [pltpu.VMEM((_EMBED_DIM, 129), jnp.float32) for _ in range(_CRING)]
            + [pltpu.VMEM((2048,), jnp.float32) for _ in range(_CRING)]
            + [pltpu.SemaphoreType.DMA for _ in range(2 * _CRING)]
        ),
    )
    def convert(tT_hbm, tail_hbm, out_hbm, *scr):
        bufs = scr[:_CRING]
        obufs = scr[_CRING:2 * _CRING]
        sins = scr[2 * _CRING:3 * _CRING]
        souts = scr[3 * _CRING:]
        wid = lax.axis_index("s") * _NC + lax.axis_index("c")
        start = wid * _TCQ + jnp.minimum(wid, _TCR)
        count = jnp.where(wid < _TCR, _TCQ + 1, _TCQ)
        lanes = lax.iota(jnp.int32, _L)

        @pl.when(wid == 0)
        def _():
            pltpu.sync_copy(tail_hbm, obufs[0].at[pl.ds(0, 1024)])
            pltpu.sync_copy(obufs[0].at[pl.ds(0, 1024)],
                            out_hbm.at[pl.ds(_TC_FULL * 128 * 16, 1024)])

        def fire_in(tc, buf, sem):
            pltpu.make_async_copy(
                tT_hbm.at[:, pl.ds(tc * 128, 128)], buf.at[:, pl.ds(0, 128)], sem
            ).start()

        def wait_in(buf, sem):
            pltpu.make_async_copy(
                tT_hbm.at[:, pl.ds(0, 128)], buf.at[:, pl.ds(0, 128)], sem
            ).wait()

        for r in range(_CRING):
            fire_in(start + r, bufs[r], sins[r])

        def round_body(g, carry):
            for r in range(_CRING):
                i = g * _CRING + r

                @pl.when(i < count)
                def _(r=r, i=i):
                    buf, obuf, sin, sout = bufs[r], obufs[r], sins[r], souts[r]
                    wait_in(buf, sin)

                    @pl.when(i >= _CRING)
                    def _():
                        pltpu.make_async_copy(
                            obuf, out_hbm.at[pl.ds(0, 2048)], sout
                        ).wait()

                    for jb in range(0, 128, 8):
                        vs = [
                            plsc.load_gather(
                                buf, [lanes, jnp.full((_L,), jb + t)]
                            )
                            for t in range(8)
                        ]
                        for t in range(8):
                            obuf[pl.ds((jb + t) * _L, _L)] = vs[t]
                    tc = start + i
                    pltpu.make_async_copy(
                        obuf, out_hbm.at[pl.ds(tc * 2048, 2048)], sout
                    ).start()

                    @pl.when(i + _CRING < count)
                    def _():
                        fire_in(tc + _CRING, buf, sin)

            return carry

        lax.fori_loop(0, (count + _CRING - 1) // _CRING, round_body, 0)

        for r in range(_CRING):
            pltpu.make_async_copy(
                obufs[r], out_hbm.at[pl.ds(0, 2048)], souts[r]
            ).wait()

    return convert(tT, tail).reshape(_NROWS, _EMBED_DIM)


def kernel(x, table):
    table_rm = _to_row_major(table)
    x_flat = x.reshape(-1)
    off = jnp.asarray(np.tile(_OFFSETS, _CB))

    @functools.partial(
        pl.kernel,
        mesh=plsc.VectorSubcoreMesh(**_MESH),
        out_type=jax.ShapeDtypeStruct((_BATCH,), jnp.float32),
        compiler_params=pltpu.CompilerParams(
            needs_layout_passes=False, use_tc_tiling_on_sc=False
        ),
        scratch_types=[
            pltpu.VMEM((_XLEN,), jnp.int32),
            pltpu.VMEM((_ROWS,), jnp.int32),
            pltpu.VMEM((_ROWS, _EMBED_DIM), jnp.float32),
            pltpu.VMEM((_ROWS, _EMBED_DIM), jnp.float32),
            pltpu.VMEM((_BPW,), jnp.float32),
            pltpu.SemaphoreType.DMA,
            pltpu.SemaphoreType.DMA,
        ],
    )
    def fm_kernel(x_hbm, off_hbm, table_hbm, out_hbm,
                  idxall, offv, rows0, rows1, zbuf, sem0, sem1):
        wid = lax.axis_index("s") * _NC + lax.axis_index("c")
        base_flat = wid * _XLEN
        pltpu.sync_copy(x_hbm.at[pl.ds(base_flat, _XLEN)], idxall)
        pltpu.sync_copy(off_hbm, offv)
        lanes = lax.iota(jnp.int32, _L)
        ring = ((rows0, sem0), (rows1, sem1))

        def off_body(c, carry):
            b0 = c * _ROWS
            for i in range(_ROWS // _L):
                ii = jnp.full((_L,), b0 + i * _L) + lanes
                v = plsc.load_gather(idxall, [ii])
                plsc.store_scatter(idxall, [ii], v + offv[pl.ds(i * _L, _L)])
            return carry

        lax.fori_loop(0, _NCHUNK, off_body, 0)

        def fire(c, rows_ref, sem):
            b0 = c * _ROWS
            for g in range(_NG):
                pltpu.make_async_copy(
                    table_hbm.at[idxall.at[pl.ds(b0 + g * _GSUB, _GSUB)]],
                    rows_ref.at[pl.ds(g * _GSUB, _GSUB), :],
                    sem,
                ).start()

        def drain(rows_ref, sem):
            pltpu.make_async_copy(
                table_hbm.at[pl.ds(0, _ROWS), :], rows_ref, sem
            ).wait()

        fire(0, rows0, sem0)
        fire(1, rows1, sem1)

        def round_body(g, carry):
            for r in range(_RING):
                c = g * _RING + r
                rows_ref, sem = ring[r]
                drain(rows_ref, sem)
                zvec = jnp.zeros((_L,), jnp.float32)
                for b in range(_CB):
                    r0 = b * _NUM_FIELDS
                    v = rows_ref[r0, :]
                    s = v
                    sq = v * v
                    for f in range(1, _NUM_FIELDS):
                        v = rows_ref[r0 + f, :]
                        s = s + v
                        sq = sq + v * v
                    t = (s * s - sq) * 0.5
                    z = jnp.sum(t)
                    zvec = jnp.where(lanes == b, jnp.full((_L,), z), zvec)
                plsc.store_scatter(zbuf, [jnp.full((_L,), c * _CB) + lanes], zvec)
                c2 = c + _RING

                @pl.when(c2 < _NCHUNK)
                def _():
                    fire(c2, rows_ref, sem)

            return carry

        lax.fori_loop(0, _NCHUNK // _RING, round_body, 0)

        for i in range(_BPW // _L):
            sl = pl.ds(i * _L, _L)
            v = zbuf[sl]
            zbuf[sl] = 1.0 / (1.0 + jnp.exp(-v))
        pltpu.sync_copy(zbuf, out_hbm.at[pl.ds(wid * _BPW, _BPW)])

    return fm_kernel(x_flat, off, table_rm)


# speedup vs baseline: 1.1653x; 1.1653x over previous
import functools

import numpy as np
import jax
import jax.numpy as jnp
from jax import lax
from jax.experimental import pallas as pl
from jax.experimental.pallas import tpu as pltpu
from jax.experimental.pallas import tpu_sc as plsc

_NUM_FIELDS = 26
_EMBED_DIM = 16
_BATCH = 16384
_FIELD_DIM = 100000
_NROWS = _FIELD_DIM * _NUM_FIELDS
_OFFSETS = np.arange(_NUM_FIELDS, dtype=np.int32) * _FIELD_DIM

_NC = 2
_NS = 16
_NW = _NC * _NS
_L = 16

_TC_FULL = _NROWS // 128
_TAILROWS = _NROWS - _TC_FULL * 128
_TCQ, _TCR = divmod(_TC_FULL, _NW)
_CRING = 8

_BPW = _BATCH // _NW
_CB = 16
_ROWS = _CB * _NUM_FIELDS
_NCHUNK = _BPW // _CB
_GSUB = 104
_NG = _ROWS // _GSUB
_XLEN = _BPW * _NUM_FIELDS
_RING = 2

_MESH = dict(core_axis_name="c", subcore_axis_name="s")


def _to_row_major(table):
    tT = table.T
    tail = table[_TC_FULL * 128:, :].reshape(-1)

    @functools.partial(
        pl.kernel,
        mesh=plsc.VectorSubcoreMesh(**_MESH),
        out_type=jax.ShapeDtypeStruct((_NROWS * _EMBED_DIM,), jnp.float32),
        compiler_params=pltpu.CompilerParams(
            needs_layout_passes=False, use_tc_tiling_on_sc=True
        ),
        scratch_types=(
            [pltpu.VMEM((_EMBED_DIM, 137), jnp.float32) for _ in range(_CRING)]
            + [pltpu.VMEM((2048,), jnp.float32) for _ in range(_CRING)]
            + [pltpu.SemaphoreType.DMA for _ in range(2 * _CRING)]
        ),
    )
    def convert(tT_hbm, tail_hbm, out_hbm, *scr):
        bufs = scr[:_CRING]
        obufs = scr[_CRING:2 * _CRING]
        sins = scr[2 * _CRING:3 * _CRING]
        souts = scr[3 * _CRING:]
        wid = lax.axis_index("s") * _NC + lax.axis_index("c")
        start = wid * _TCQ + jnp.minimum(wid, _TCR)
        count = jnp.where(wid < _TCR, _TCQ + 1, _TCQ)
        lanes = lax.iota(jnp.int32, _L)

        @pl.when(wid == 0)
        def _():
            pltpu.sync_copy(tail_hbm, obufs[0].at[pl.ds(0, 1024)])
            pltpu.sync_copy(obufs[0].at[pl.ds(0, 1024)],
                            out_hbm.at[pl.ds(_TC_FULL * 128 * 16, 1024)])

        def fire_in(tc, buf, sem):
            pltpu.make_async_copy(
                tT_hbm.at[:, pl.ds(tc * 128, 128)], buf.at[:, pl.ds(0, 128)], sem
            ).start()

        def wait_in(buf, sem):
            pltpu.make_async_copy(
                tT_hbm.at[:, pl.ds(0, 128)], buf.at[:, pl.ds(0, 128)], sem
            ).wait()

        for r in range(_CRING):
            fire_in(start + r, bufs[r], sins[r])

        def round_body(g, carry):
            for r in range(_CRING):
                i = g * _CRING + r

                @pl.when(i < count)
                def _(r=r, i=i):
                    buf, obuf, sin, sout = bufs[r], obufs[r], sins[r], souts[r]
                    wait_in(buf, sin)

                    @pl.when(i >= _CRING)
                    def _():
                        pltpu.make_async_copy(
                            obuf, out_hbm.at[pl.ds(0, 2048)], sout
                        ).wait()

                    @plsc.parallel_loop(0, 128, step=1, unroll=8)
                    def _(j):
                        v = plsc.load_gather(buf, [lanes, jnp.full((_L,), j)])
                        obuf[pl.ds(j * _L, _L)] = v
                    tc = start + i
                    pltpu.make_async_copy(
                        obuf, out_hbm.at[pl.ds(tc * 2048, 2048)], sout
                    ).start()

                    @pl.when(i + _CRING < count)
                    def _():
                        fire_in(tc + _CRING, buf, sin)

            return carry

        lax.fori_loop(0, (count + _CRING - 1) // _CRING, round_body, 0)

        for r in range(_CRING):
            pltpu.make_async_copy(
                obufs[r], out_hbm.at[pl.ds(0, 2048)], souts[r]
            ).wait()

    return convert(tT, tail).reshape(_NROWS, _EMBED_DIM)


def kernel(x, table):
    table_rm = _to_row_major(table)
    x_flat = x.reshape(-1)
    off = jnp.asarray(np.tile(_OFFSETS, _CB))

    @functools.partial(
        pl.kernel,
        mesh=plsc.VectorSubcoreMesh(**_MESH),
        out_type=jax.ShapeDtypeStruct((_BATCH,), jnp.float32),
        compiler_params=pltpu.CompilerParams(
            needs_layout_passes=False, use_tc_tiling_on_sc=False
        ),
        scratch_types=[
            pltpu.VMEM((_XLEN,), jnp.int32),
            pltpu.VMEM((_ROWS,), jnp.int32),
            pltpu.VMEM((_ROWS, _EMBED_DIM), jnp.float32),
            pltpu.VMEM((_ROWS, _EMBED_DIM), jnp.float32),
            pltpu.VMEM((_BPW,), jnp.float32),
            pltpu.SemaphoreType.DMA,
            pltpu.SemaphoreType.DMA,
        ],
    )
    def fm_kernel(x_hbm, off_hbm, table_hbm, out_hbm,
                  idxall, offv, rows0, rows1, zbuf, sem0, sem1):
        wid = lax.axis_index("s") * _NC + lax.axis_index("c")
        base_flat = wid * _XLEN
        pltpu.sync_copy(x_hbm.at[pl.ds(base_flat, _XLEN)], idxall)
        pltpu.sync_copy(off_hbm, offv)
        lanes = lax.iota(jnp.int32, _L)
        ring = ((rows0, sem0), (rows1, sem1))

        def off_body(c, carry):
            b0 = c * _ROWS
            for i in range(_ROWS // _L):
                ii = jnp.full((_L,), b0 + i * _L) + lanes
                v = plsc.load_gather(idxall, [ii])
                plsc.store_scatter(idxall, [ii], v + offv[pl.ds(i * _L, _L)])
            return carry

        lax.fori_loop(0, _NCHUNK, off_body, 0)

        def fire(c, rows_ref, sem):
            b0 = c * _ROWS
            for g in range(_NG):
                pltpu.make_async_copy(
                    table_hbm.at[idxall.at[pl.ds(b0 + g * _GSUB, _GSUB)]],
                    rows_ref.at[pl.ds(g * _GSUB, _GSUB), :],
                    sem,
                ).start()

        def drain(rows_ref, sem):
            pltpu.make_async_copy(
                table_hbm.at[pl.ds(0, _ROWS), :], rows_ref, sem
            ).wait()

        fire(0, rows0, sem0)
        fire(1, rows1, sem1)

        def round_body(g, carry):
            for r in range(_RING):
                c = g * _RING + r
                rows_ref, sem = ring[r]
                drain(rows_ref, sem)
                zvec = jnp.zeros((_L,), jnp.float32)
                for b in range(_CB):
                    r0 = b * _NUM_FIELDS
                    v = rows_ref[r0, :]
                    s = v
                    sq = v * v
                    for f in range(1, _NUM_FIELDS):
                        v = rows_ref[r0 + f, :]
                        s = s + v
                        sq = sq + v * v
                    t = (s * s - sq) * 0.5
                    z = jnp.sum(t)
                    zvec = jnp.where(lanes == b, jnp.full((_L,), z), zvec)
                plsc.store_scatter(zbuf, [jnp.full((_L,), c * _CB) + lanes], zvec)
                c2 = c + _RING

                @pl.when(c2 < _NCHUNK)
                def _():
                    fire(c2, rows_ref, sem)

            return carry

        lax.fori_loop(0, _NCHUNK // _RING, round_body, 0)

        for i in range(_BPW // _L):
            sl = pl.ds(i * _L, _L)
            v = zbuf[sl]
            zbuf[sl] = 1.0 / (1.0 + jnp.exp(-v))
        pltpu.sync_copy(zbuf, out_hbm.at[pl.ds(wid * _BPW, _BPW)])

    return fm_kernel(x_flat, off, table_rm)
